# baseline (speedup 1.0000x reference)
import functools

import jax
import jax.numpy as jnp
from jax import lax
from jax.experimental import pallas as pl
from jax.experimental.pallas import tpu as pltpu
from jax.experimental.pallas import tpu_sc as plsc

B = 16384
HID = 1024
NC, NS, L = 2, 16, 16
NW = NC * NS
BW = B // NW
CHUNK = 128
NJ = BW // CHUNK
TROWS = 88

TILE = 512
GRID = B // TILE


@functools.cache
def _sc_gather_kernel():
    mesh = plsc.VectorSubcoreMesh(core_axis_name="c", subcore_axis_name="s")

    @functools.partial(
        pl.kernel,
        mesh=mesh,
        out_type=jax.ShapeDtypeStruct((NW, BW, 128), jnp.float32),
        scratch_types=[
            pltpu.VMEM((NJ, CHUNK), jnp.int32),
            pltpu.VMEM((NJ, CHUNK), jnp.int32),
            pltpu.VMEM((NJ, CHUNK), jnp.int32),
            pltpu.VMEM((BW, 128), jnp.float32),
            pltpu.SemaphoreType.DMA,
        ],
    )
    def _sc_gather(i0_hbm, i1_hbm, table_hbm, x_hbm,
                   idx0_v, idx1_v, cidx_v, rows_v, sem):
        wid = lax.axis_index("s") * NC + lax.axis_index("c")
        pltpu.sync_copy(i0_hbm.at[wid], idx0_v)
        pltpu.sync_copy(i1_hbm.at[wid], idx1_v)
        for j in range(NJ):
            for k in range(CHUNK // L):
                s = pl.ds(k * L, L)
                cidx_v[j, s] = idx0_v[j, s] * 12 + idx1_v[j, s]
        copies = []
        for j in range(NJ):
            copies.append(pltpu.async_copy(
                table_hbm.at[cidx_v.at[j]],
                rows_v.at[pl.ds(j * CHUNK, CHUNK)], sem))
        for c in copies:
            c.wait()
        pltpu.sync_copy(rows_v, x_hbm.at[wid])

    return _sc_gather


def _mlp_body(x_ref, bin_ref, w1_ref, w1c_ref, b1_ref, w2_ref, b2_ref,
              out_ref):
    h = jnp.dot(x_ref[...].astype(jnp.bfloat16), w1_ref[...],
                preferred_element_type=jnp.float32)
    h += bin_ref[:, 0:1] * w1c_ref[0:1, :]
    h += bin_ref[:, 1:2] * w1c_ref[1:2, :]
    h += b1_ref[...]
    h = h * (1.0 / (1.0 + jnp.exp(-h)))
    out = jnp.dot(h.astype(jnp.bfloat16), w2_ref[...],
                  preferred_element_type=jnp.float32)
    out_ref[...] = out + b2_ref[...]


def _mlp_call(x, bin2, w1p, w1c, b1r, w2, b2r):
    full = lambda s: pl.BlockSpec(s, lambda i: (0, 0))
    return pl.pallas_call(
        _mlp_body,
        grid=(GRID,),
        in_specs=[
            pl.BlockSpec((TILE, 128), lambda i: (i, 0)),
            pl.BlockSpec((TILE, 2), lambda i: (i, 0)),
            full((128, HID)),
            full((2, HID)),
            full((1, HID)),
            full((HID, HID)),
            full((1, HID)),
        ],
        out_specs=pl.BlockSpec((TILE, HID), lambda i: (i, 0)),
        out_shape=jax.ShapeDtypeStruct((B, HID), jnp.float32),
    )(x, bin2, w1p, w1c, b1r, w2, b2r)


def kernel(cal, dow_emb, month_emb, W1, b1, W2, b2):
    cal = cal.astype(jnp.int32)
    i0 = cal[:, 0].reshape(NW, NJ, CHUNK)
    i1 = cal[:, 1].reshape(NW, NJ, CHUNK)
    bin2 = cal[:, 2:4].astype(jnp.float32)

    cd = jnp.broadcast_to(dow_emb[:, None, :], (7, 12, 16)).reshape(84, 16)
    cm = jnp.broadcast_to(month_emb[None, :, :], (7, 12, 16)).reshape(84, 16)
    table = jnp.concatenate(
        [cd, cm, jnp.zeros((84, 96), jnp.float32)], axis=1)
    table = jnp.concatenate(
        [table, jnp.zeros((TROWS - 84, 128), jnp.float32)], axis=0)

    x = _sc_gather_kernel()(i0, i1, table).reshape(B, 128)

    w1p = jnp.concatenate(
        [W1, jnp.zeros((128 - 34, HID), jnp.float32)], axis=0
    ).astype(jnp.bfloat16)
    w1c = W1[32:34]
    b1r = b1.reshape(1, HID)
    b2r = b2.reshape(1, HID)
    return _mlp_call(x, bin2, w1p, w1c, b1r, W2.astype(jnp.bfloat16), b2r)


# speedup vs baseline: 1.1101x; 1.1101x over previous
import functools

import jax
import jax.numpy as jnp
from jax import lax
from jax.experimental import pallas as pl
from jax.experimental.pallas import tpu as pltpu
from jax.experimental.pallas import tpu_sc as plsc

B = 16384
HID = 1024
NC, NS, L = 2, 16, 16
NW = NC * NS
BW = B // NW
CHUNK = 128
NJ = BW // CHUNK
TROWS = 4120

TILE = 1024
GRID = B // TILE


@functools.cache
def _sc_gather_kernel():
    mesh = plsc.VectorSubcoreMesh(core_axis_name="c", subcore_axis_name="s")

    @functools.partial(
        pl.kernel,
        mesh=mesh,
        out_type=jax.ShapeDtypeStruct((B, 128), jnp.float32),
        scratch_types=[
            pltpu.VMEM((NJ, CHUNK), jnp.int32),
            pltpu.VMEM((NJ, CHUNK), jnp.int32),
            pltpu.VMEM((NJ, CHUNK), jnp.int32),
            pltpu.VMEM((NJ, CHUNK), jnp.int32),
            pltpu.VMEM((NJ, CHUNK), jnp.int32),
            pltpu.VMEM((BW, 128), jnp.float32),
            pltpu.SemaphoreType.DMA,
        ],
    )
    def _sc_gather(i0_hbm, i1_hbm, i2_hbm, i3_hbm, table_hbm, x_hbm,
                   idx0_v, idx1_v, idx2_v, idx3_v, cidx_v, rows_v, sem):
        wid = lax.axis_index("s") * NC + lax.axis_index("c")
        pltpu.sync_copy(i0_hbm.at[wid], idx0_v)
        pltpu.sync_copy(i1_hbm.at[wid], idx1_v)
        pltpu.sync_copy(i2_hbm.at[wid], idx2_v)
        pltpu.sync_copy(i3_hbm.at[wid], idx3_v)
        for j in range(NJ):
            for k in range(CHUNK // L):
                s = pl.ds(k * L, L)
                cidx_v[j, s] = ((idx0_v[j, s] * 12 + idx1_v[j, s]) * 49
                                + idx2_v[j, s] * 7 + idx3_v[j, s])
        copies = []
        for j in range(NJ):
            copies.append(pltpu.async_copy(
                table_hbm.at[cidx_v.at[j]],
                rows_v.at[pl.ds(j * CHUNK, CHUNK)], sem))
        for c in copies:
            c.wait()
        pltpu.sync_copy(rows_v, x_hbm.at[pl.ds(wid * BW, BW)])

    return _sc_gather


def _mlp_body(x_ref, w1_ref, w2_ref, b2_ref, out_ref):
    h = jnp.dot(x_ref[...].astype(jnp.bfloat16), w1_ref[...],
                preferred_element_type=jnp.float32)
    h = h * (1.0 / (1.0 + jnp.exp(-h)))
    out = jnp.dot(h.astype(jnp.bfloat16), w2_ref[...],
                  preferred_element_type=jnp.float32)
    out_ref[...] = out + b2_ref[...]


def _mlp_call(x, w1p, w2, b2r):
    full = lambda s: pl.BlockSpec(s, lambda i: (0, 0))
    return pl.pallas_call(
        _mlp_body,
        grid=(GRID,),
        in_specs=[
            pl.BlockSpec((TILE, 128), lambda i: (i, 0)),
            full((128, HID)),
            full((HID, HID)),
            full((1, HID)),
        ],
        out_specs=pl.BlockSpec((TILE, HID), lambda i: (i, 0)),
        out_shape=jax.ShapeDtypeStruct((B, HID), jnp.float32),
    )(x, w1p, w2, b2r)


def kernel(cal, dow_emb, month_emb, W1, b1, W2, b2):
    cal = cal.astype(jnp.int32)
    i0 = cal[:, 0].reshape(NW, NJ, CHUNK)
    i1 = cal[:, 1].reshape(NW, NJ, CHUNK)
    i2 = cal[:, 2].reshape(NW, NJ, CHUNK)
    i3 = cal[:, 3].reshape(NW, NJ, CHUNK)

    f = jnp.arange(7, dtype=jnp.float32)
    cd = jnp.broadcast_to(dow_emb[:, None, None, None, :], (7, 12, 7, 7, 16))
    cm = jnp.broadcast_to(month_emb[None, :, None, None, :], (7, 12, 7, 7, 16))
    c2 = jnp.broadcast_to(f[None, None, :, None, None], (7, 12, 7, 7, 1))
    c3 = jnp.broadcast_to(f[None, None, None, :, None], (7, 12, 7, 7, 1))
    ones = jnp.ones((7, 12, 7, 7, 1), jnp.float32)
    zeros = jnp.zeros((7, 12, 7, 7, 93), jnp.float32)
    table = jnp.concatenate([cd, cm, c2, c3, ones, zeros],
                            axis=-1).reshape(4116, 128)
    table = jnp.concatenate(
        [table, jnp.zeros((TROWS - 4116, 128), jnp.float32)], axis=0)

    x = _sc_gather_kernel()(i0, i1, i2, i3, table)

    w1p = jnp.concatenate(
        [W1, b1.reshape(1, HID), jnp.zeros((128 - 35, HID), jnp.float32)],
        axis=0).astype(jnp.bfloat16)
    b2r = b2.reshape(1, HID)
    return _mlp_call(x, w1p, W2.astype(jnp.bfloat16), b2r)
